# baseline (speedup 1.0000x reference)
import functools

import jax
import jax.numpy as jnp
from jax import lax
from jax.experimental import pallas as pl
from jax.experimental.pallas import tpu as pltpu
from jax.experimental.pallas import tpu_sc as plsc

N = 100000
C = 64
K = 27
BLK = 1024
NB = 98
NP = NB * BLK
NQ8 = NP // 8
EP = NP + 8
FP = N + 256
ECH = 2000
EB = 400
NEC = N // EB
FSEG = 256
HB = 256
NHB = NP // HB
KS0 = 14
SINK = 2 ** 30
MAXB1 = -(-(KS0 * NEC) // 16)
MAXB2 = -(-(KS0 * NHB) // 16)


def _sc_invert_gather(feats_pad, src, dst):
  mesh = plsc.VectorSubcoreMesh(core_axis_name="c", subcore_axis_name="s")

  @functools.partial(
      pl.kernel,
      out_type=(jax.ShapeDtypeStruct((NB, K, BLK, C), jnp.float32),
                jax.ShapeDtypeStruct((K * NP,), jnp.int32),
                jax.ShapeDtypeStruct((K * EP, C), jnp.float32)),
      mesh=mesh,
      compiler_params=pltpu.CompilerParams(
          needs_layout_passes=False, use_tc_tiling_on_sc=False),
      scratch_types=[
          pltpu.VMEM((NQ8,), jnp.int32),
          pltpu.VMEM((ECH,), jnp.int32),
          pltpu.VMEM((ECH,), jnp.int32),
          pltpu.VMEM((EB,), jnp.int32),
          pltpu.VMEM((EB,), jnp.int32),
          pltpu.VMEM((FSEG, C), jnp.float32),
          pltpu.VMEM((EB, C), jnp.float32),
          pltpu.VMEM((HB,), jnp.int32),
          pltpu.VMEM((HB + 8, C), jnp.float32),
          pltpu.VMEM((HB, C), jnp.float32),
          pltpu.SMEM((32,), jnp.int32),
          pltpu.SMEM((32,), jnp.int32),
      ],
  )
  def sc_kernel(feats_hbm, src_hbm, dst_hbm, g_hbm, inv_hbm, a_hbm,
                inv_v, dbufa, sbufa, dbufb, sbufb, fbuf, abuf,
                ivbuf, awin, gbuf, gmin_s, gmax_s):
    cid = lax.axis_index("c")
    sid = lax.axis_index("s")
    kbase = cid * KS0
    nk = KS0 - cid
    iota = lax.iota(jnp.int32, 16)

    @pl.when(sid < nk)
    def _build():
      k = kbase + sid
      for h in range(8):
        lo = h * NQ8

        @pl.loop(0, NQ8 // 16)
        def _init(i):
          inv_v[pl.ds(i * 16, 16)] = jnp.full((16,), SINK, jnp.int32)

        @pl.loop(0, N // ECH)
        def _chunk(j):
          e0 = pl.multiple_of(k * N + j * ECH, 8)
          pltpu.sync_copy(dst_hbm.at[pl.ds(e0, ECH)], dbufa)

          @pl.loop(0, ECH // 16)
          def _scatter(i):
            dv = dbufa[pl.ds(i * 16, 16)]
            ev = jnp.full((16,), j * ECH + i * 16, jnp.int32) + iota
            m = (dv >= lo) & (dv < lo + NQ8) & (dv < N)
            iv = jnp.where(m, dv - lo, jnp.zeros((16,), jnp.int32))
            plsc.store_scatter(inv_v, [iv], ev, mask=m)

        pltpu.sync_copy(
            inv_v, inv_hbm.at[pl.ds(pl.multiple_of(k * NP + lo, 8), NQ8)])

    plsc.subcore_barrier()

    @pl.loop(0, MAXB1)
    def _b1(t):
      j = t * 16 + sid

      @pl.when(j < nk * NEC)
      def _item():
        k_i = j // NEC
        k = kbase + k_i
        e0 = (j - k_i * NEC) * EB
        base = pl.multiple_of(k * N + e0, 8)
        pltpu.sync_copy(dst_hbm.at[pl.ds(base, EB)], dbufb)
        pltpu.sync_copy(src_hbm.at[pl.ds(base, EB)], sbufb)

        def _group_mm(g, carry):
          glo, ghi = carry
          dv = dbufb[pl.ds(g * 16, 16)]
          sv = sbufb[pl.ds(g * 16, 16)]
          m = dv < N
          mn = jnp.min(jnp.where(m, sv, jnp.full((16,), SINK, jnp.int32)))
          mx = jnp.max(jnp.where(m, sv, jnp.full((16,), -1, jnp.int32)))
          gmin_s[g] = mn
          gmax_s[g] = mx
          return (jnp.minimum(glo, mn), jnp.maximum(ghi, mx))

        lo, hi = lax.fori_loop(0, EB // 16, _group_mm,
                               (jnp.int32(SINK), jnp.int32(-1)))

        @pl.when(hi >= 0)
        def _segs():
          seg0 = lo & ~jnp.int32(7)
          nseg = (hi - seg0) // FSEG + 1

          def _seg(p, _):
            sbase = pl.multiple_of(seg0 + p * FSEG, 8)
            pltpu.sync_copy(feats_hbm.at[pl.ds(sbase, FSEG)], fbuf)

            def _group(g, _):
              @pl.when((gmax_s[g] >= sbase) & (gmin_s[g] < sbase + FSEG))
              def _hit():
                dv = dbufb[pl.ds(g * 16, 16)]
                sv = sbufb[pl.ds(g * 16, 16)]
                m = (dv < N) & (sv >= sbase) & (sv < sbase + FSEG)
                rel = jnp.where(m, sv - sbase, jnp.zeros((16,), jnp.int32))
                rows = jnp.full((16,), g * 16, jnp.int32) + iota
                for w in range(C):
                  wv = jnp.full((16,), w, jnp.int32)
                  vals = plsc.load_gather(fbuf, [rel, wv])
                  plsc.store_scatter(abuf, [rows, wv], vals, mask=m)
              return 0

            lax.fori_loop(0, EB // 16, _group, 0)
            return 0

          lax.fori_loop(0, nseg, _seg, 0)

        pltpu.sync_copy(
            abuf, a_hbm.at[pl.ds(pl.multiple_of(k * EP + e0, 8), EB)])

    plsc.subcore_barrier()

    @pl.loop(0, MAXB2)
    def _b2(t):
      j = t * 16 + sid

      @pl.when(j < nk * NHB)
      def _item():
        k_i = j // NHB
        k = kbase + k_i
        r = j - k_i * NHB
        row0 = r * HB
        b = row0 // BLK
        ri = pl.multiple_of(row0 - b * BLK, 8)
        pltpu.sync_copy(
            inv_hbm.at[pl.ds(pl.multiple_of(k * NP + row0, 8), HB)], ivbuf)

        def _mm(g, carry):
          glo, ghi = carry
          iv = ivbuf[pl.ds(g * 16, 16)]
          m = iv < SINK
          mn = jnp.min(jnp.where(m, iv, jnp.full((16,), SINK, jnp.int32)))
          mx = jnp.max(jnp.where(m, iv, jnp.full((16,), -1, jnp.int32)))
          return (jnp.minimum(glo, mn), jnp.maximum(ghi, mx))

        lo, hi = lax.fori_loop(0, HB // 16, _mm,
                               (jnp.int32(SINK), jnp.int32(-1)))
        lo8 = lo & ~jnp.int32(7)

        @pl.when(hi >= 0)
        def _load():
          pltpu.sync_copy(
              a_hbm.at[pl.ds(pl.multiple_of(k * EP + lo8, 8), HB + 8)],
              awin)

        @pl.loop(0, HB // 16)
        def _group(g):
          iv = ivbuf[pl.ds(g * 16, 16)]
          m = iv < SINK
          rel = jnp.where(m, iv - lo8, jnp.zeros((16,), jnp.int32))
          rows = jnp.full((16,), g * 16, jnp.int32) + iota
          for w in range(C):
            wv = jnp.full((16,), w, jnp.int32)
            vals = plsc.load_gather(awin, [rel, wv])
            vals = jnp.where(m, vals, jnp.zeros((16,), jnp.float32))
            plsc.store_scatter(gbuf, [rows, wv], vals)

        pltpu.sync_copy(gbuf, g_hbm.at[b, k, pl.ds(ri, HB)])

  return sc_kernel(feats_pad, src, dst)[0]


def _tc_gemm(g, wflat):
  def body(g_ref, w_ref, o_ref, x_ref):
    for k in range(K):
      x_ref[:, k * C:(k + 1) * C] = g_ref[0, k, :, :]
    o_ref[...] = jnp.dot(x_ref[...], w_ref[...],
                         preferred_element_type=jnp.float32)

  return pl.pallas_call(
      body,
      grid=(NB,),
      in_specs=[
          pl.BlockSpec((1, K, BLK, C), lambda b: (b, 0, 0, 0)),
          pl.BlockSpec((K * C, C), lambda b: (0, 0)),
      ],
      out_specs=pl.BlockSpec((BLK, C), lambda b: (b, 0)),
      out_shape=jax.ShapeDtypeStruct((NP, C), jnp.float32),
      scratch_shapes=[pltpu.VMEM((BLK, K * C), jnp.float32)],
      compiler_params=pltpu.CompilerParams(
          dimension_semantics=("parallel",)),
  )(g, wflat)


def _tc_stats(out_pre):
  def body(o_ref, st_ref):
    x = o_ref[...]
    s = jnp.sum(x, axis=0, keepdims=True)
    q = jnp.sum(x * x, axis=0, keepdims=True)
    z = jnp.zeros((1, C), jnp.float32)
    tile = jnp.concatenate(
        [jnp.concatenate([s, z], axis=1),
         jnp.concatenate([q, z], axis=1),
         jnp.zeros((6, 128), jnp.float32)], axis=0)

    @pl.when(pl.program_id(0) == 0)
    def _():
      st_ref[...] = tile

    @pl.when(pl.program_id(0) != 0)
    def _():
      st_ref[...] += tile

  return pl.pallas_call(
      body,
      grid=(NB,),
      in_specs=[pl.BlockSpec((BLK, C), lambda b: (b, 0))],
      out_specs=pl.BlockSpec((8, 128), lambda b: (0, 0)),
      out_shape=jax.ShapeDtypeStruct((8, 128), jnp.float32),
      compiler_params=pltpu.CompilerParams(
          dimension_semantics=("arbitrary",)),
  )(out_pre)


def _tc_bn_relu(out_pre, stats, gamma8, beta8):
  def body(o_ref, st_ref, ga_ref, be_ref, out_ref):
    s = st_ref[0:1, 0:C]
    q = st_ref[1:2, 0:C]
    mean = s * (1.0 / N)
    var = q * (1.0 / N) - mean * mean
    inv = lax.rsqrt(var + 1e-5)
    scale = ga_ref[0:1, :] * inv
    shift = be_ref[0:1, :] - mean * scale
    out_ref[...] = jnp.maximum(o_ref[...] * scale + shift, 0.0)

  return pl.pallas_call(
      body,
      grid=(NB,),
      in_specs=[
          pl.BlockSpec((BLK, C), lambda b: (b, 0)),
          pl.BlockSpec((8, 128), lambda b: (0, 0)),
          pl.BlockSpec((8, C), lambda b: (0, 0)),
          pl.BlockSpec((8, C), lambda b: (0, 0)),
      ],
      out_specs=pl.BlockSpec((BLK, C), lambda b: (b, 0)),
      out_shape=jax.ShapeDtypeStruct((NP, C), jnp.float32),
      compiler_params=pltpu.CompilerParams(
          dimension_semantics=("parallel",)),
  )(out_pre, stats, gamma8, beta8)


def kernel(feats, W, gamma, beta, src, dst):
  feats_pad = jnp.concatenate(
      [feats, jnp.zeros((FP - N, C), jnp.float32)], axis=0)
  src_flat = src.reshape(K * N)
  dst_flat = dst.reshape(K * N)
  wflat = W.reshape(K * C, C)
  gamma8 = jnp.broadcast_to(gamma[None, :], (8, C))
  beta8 = jnp.broadcast_to(beta[None, :], (8, C))

  g = _sc_invert_gather(feats_pad, src_flat, dst_flat)
  out_pre = _tc_gemm(g, wflat)
  stats = _tc_stats(out_pre)
  out = _tc_bn_relu(out_pre, stats, gamma8, beta8)
  return out[:N]


# speedup vs baseline: 1.0459x; 1.0459x over previous
import functools

import jax
import jax.numpy as jnp
from jax import lax
from jax.experimental import pallas as pl
from jax.experimental.pallas import tpu as pltpu
from jax.experimental.pallas import tpu_sc as plsc

N = 100000
C = 64
K = 27
BLK = 1024
NB = 98
NP = NB * BLK
NQ8 = NP // 8
EP = NP + 8
FP = N + 256
ECH = 2000
EB = 400
NEC = N // EB
FSEG = 224
HB = 256
NHB = NP // HB
KS0 = 14
SINK = 2 ** 30
MAXB1 = -(-(KS0 * NEC) // 16)
MAXB2 = -(-(KS0 * NHB) // 16)


def _sc_invert_gather(feats_pad, src, dst):
  mesh = plsc.VectorSubcoreMesh(core_axis_name="c", subcore_axis_name="s")

  @functools.partial(
      pl.kernel,
      out_type=(jax.ShapeDtypeStruct((NB, K, BLK, C), jnp.float32),
                jax.ShapeDtypeStruct((K * NP,), jnp.int32),
                jax.ShapeDtypeStruct((K * EP, C), jnp.float32)),
      mesh=mesh,
      compiler_params=pltpu.CompilerParams(
          needs_layout_passes=False, use_tc_tiling_on_sc=False),
      scratch_types=[
          pltpu.VMEM((NQ8,), jnp.int32),
          pltpu.VMEM((ECH,), jnp.int32),
          pltpu.VMEM((EB,), jnp.int32),
          pltpu.VMEM((EB,), jnp.int32),
          [pltpu.VMEM((FSEG, C), jnp.float32) for _ in range(2)],
          pltpu.VMEM((EB, C), jnp.float32),
          pltpu.VMEM((HB,), jnp.int32),
          pltpu.VMEM((HB + 8, C), jnp.float32),
          [pltpu.VMEM((HB, C), jnp.float32) for _ in range(2)],
          [pltpu.SemaphoreType.DMA for _ in range(2)],
          [pltpu.SemaphoreType.DMA for _ in range(2)],
          pltpu.SMEM((32,), jnp.int32),
          pltpu.SMEM((32,), jnp.int32),
      ],
  )
  def sc_kernel(feats_hbm, src_hbm, dst_hbm, g_hbm, inv_hbm, a_hbm,
                inv_v, dbufa, dbufb, sbufb, fbuf, abuf,
                ivbuf, awin, gbuf, lsem, wsem, gmin_s, gmax_s):
    cid = lax.axis_index("c")
    sid = lax.axis_index("s")
    kbase = cid * KS0
    nk = KS0 - cid
    iota = lax.iota(jnp.int32, 16)

    @pl.when(sid < nk)
    def _build():
      k = kbase + sid
      for h in range(8):
        lo = h * NQ8

        @pl.loop(0, NQ8 // 16)
        def _init(i):
          inv_v[pl.ds(i * 16, 16)] = jnp.full((16,), SINK, jnp.int32)

        @pl.loop(0, N // ECH)
        def _chunk(j):
          e0 = pl.multiple_of(k * N + j * ECH, 8)
          pltpu.sync_copy(dst_hbm.at[pl.ds(e0, ECH)], dbufa)

          @pl.loop(0, ECH // 16)
          def _scatter(i):
            dv = dbufa[pl.ds(i * 16, 16)]
            ev = jnp.full((16,), j * ECH + i * 16, jnp.int32) + iota
            m = (dv >= lo) & (dv < lo + NQ8) & (dv < N)
            iv = jnp.where(m, dv - lo, jnp.zeros((16,), jnp.int32))
            plsc.store_scatter(inv_v, [iv], ev, mask=m)

        pltpu.sync_copy(
            inv_v, inv_hbm.at[pl.ds(pl.multiple_of(k * NP + lo, 8), NQ8)])

    plsc.subcore_barrier()

    @pl.loop(0, MAXB1)
    def _b1(t):
      j = t * 16 + sid

      @pl.when(j < nk * NEC)
      def _item():
        k_i = j // NEC
        k = kbase + k_i
        e0 = (j - k_i * NEC) * EB
        base = pl.multiple_of(k * N + e0, 8)
        pltpu.sync_copy(dst_hbm.at[pl.ds(base, EB)], dbufb)
        pltpu.sync_copy(src_hbm.at[pl.ds(base, EB)], sbufb)

        def _group_mm(g, carry):
          glo, ghi = carry
          dv = dbufb[pl.ds(g * 16, 16)]
          sv = sbufb[pl.ds(g * 16, 16)]
          m = dv < N
          mn = jnp.min(jnp.where(m, sv, jnp.full((16,), SINK, jnp.int32)))
          mx = jnp.max(jnp.where(m, sv, jnp.full((16,), -1, jnp.int32)))
          gmin_s[g] = mn
          gmax_s[g] = mx
          return (jnp.minimum(glo, mn), jnp.maximum(ghi, mx))

        lo, hi = lax.fori_loop(0, EB // 16, _group_mm,
                               (jnp.int32(SINK), jnp.int32(-1)))

        @pl.when(hi >= 0)
        def _segs():
          seg0 = lo & ~jnp.int32(7)
          nseg = (hi - seg0) // FSEG + 1

          def _load_seg(p, slot):
            sbase = pl.multiple_of(seg0 + p * FSEG, 8)
            pltpu.async_copy(feats_hbm.at[pl.ds(sbase, FSEG)], fbuf[slot],
                             lsem[slot])

          _load_seg(0, 0)

          def _seg(p, _):
            sbase = seg0 + p * FSEG
            for slot in range(2):
              @pl.when(lax.rem(p, 2) == slot)
              def _slot(slot=slot):
                pltpu.make_async_copy(feats_hbm.at[pl.ds(0, FSEG)],
                                      fbuf[slot], lsem[slot]).wait()

                @pl.when(p + 1 < nseg)
                def _pre():
                  _load_seg(p + 1, 1 - slot)

                def _group(g, _):
                  @pl.when((gmax_s[g] >= sbase)
                           & (gmin_s[g] < sbase + FSEG))
                  def _hit():
                    dv = dbufb[pl.ds(g * 16, 16)]
                    sv = sbufb[pl.ds(g * 16, 16)]
                    m = (dv < N) & (sv >= sbase) & (sv < sbase + FSEG)
                    rel = jnp.where(m, sv - sbase,
                                    jnp.zeros((16,), jnp.int32))
                    rows = jnp.full((16,), g * 16, jnp.int32) + iota
                    for w in range(C):
                      wv = jnp.full((16,), w, jnp.int32)
                      vals = plsc.load_gather(fbuf[slot], [rel, wv])
                      plsc.store_scatter(abuf, [rows, wv], vals, mask=m)
                  return 0

                lax.fori_loop(0, EB // 16, _group, 0)
            return 0

          lax.fori_loop(0, nseg, _seg, 0)

        pltpu.sync_copy(
            abuf, a_hbm.at[pl.ds(pl.multiple_of(k * EP + e0, 8), EB)])

    plsc.subcore_barrier()

    @pl.loop(0, MAXB2)
    def _b2(t):
      j = t * 16 + sid

      @pl.when(j < nk * NHB)
      def _item():
        k_i = j // NHB
        k = kbase + k_i
        r = j - k_i * NHB
        row0 = r * HB
        b = row0 // BLK
        ri = pl.multiple_of(row0 - b * BLK, 8)
        pltpu.sync_copy(
            inv_hbm.at[pl.ds(pl.multiple_of(k * NP + row0, 8), HB)], ivbuf)

        def _mm(g, carry):
          glo, ghi = carry
          iv = ivbuf[pl.ds(g * 16, 16)]
          m = iv < SINK
          mn = jnp.min(jnp.where(m, iv, jnp.full((16,), SINK, jnp.int32)))
          mx = jnp.max(jnp.where(m, iv, jnp.full((16,), -1, jnp.int32)))
          return (jnp.minimum(glo, mn), jnp.maximum(ghi, mx))

        lo, hi = lax.fori_loop(0, HB // 16, _mm,
                               (jnp.int32(SINK), jnp.int32(-1)))
        lo8 = lo & ~jnp.int32(7)

        @pl.when(hi >= 0)
        def _load():
          pltpu.sync_copy(
              a_hbm.at[pl.ds(pl.multiple_of(k * EP + lo8, 8), HB + 8)],
              awin)

        for slot in range(2):
          @pl.when(lax.rem(t, 2) == slot)
          def _slot(slot=slot):
            @pl.when(t >= 2)
            def _():
              pltpu.make_async_copy(
                  gbuf[slot], g_hbm.at[0, 0, pl.ds(0, HB)],
                  wsem[slot]).wait()

            @pl.loop(0, HB // 16)
            def _group(g):
              iv = ivbuf[pl.ds(g * 16, 16)]
              m = iv < SINK
              rel = jnp.where(m, iv - lo8, jnp.zeros((16,), jnp.int32))
              rows = jnp.full((16,), g * 16, jnp.int32) + iota
              for w in range(C):
                wv = jnp.full((16,), w, jnp.int32)
                vals = plsc.load_gather(awin, [rel, wv])
                vals = jnp.where(m, vals, jnp.zeros((16,), jnp.float32))
                plsc.store_scatter(gbuf[slot], [rows, wv], vals)

            pltpu.async_copy(gbuf[slot], g_hbm.at[b, k, pl.ds(ri, HB)],
                             wsem[slot])

    for slot in range(2):
      pltpu.make_async_copy(
          gbuf[slot], g_hbm.at[0, 0, pl.ds(0, HB)], wsem[slot]).wait()

  return sc_kernel(feats_pad, src, dst)[0]


def _tc_gemm(g, wflat):
  def body(g_ref, w_ref, o_ref, x_ref):
    for k in range(K):
      x_ref[:, k * C:(k + 1) * C] = g_ref[0, k, :, :]
    o_ref[...] = jnp.dot(x_ref[...], w_ref[...],
                         preferred_element_type=jnp.float32)

  return pl.pallas_call(
      body,
      grid=(NB,),
      in_specs=[
          pl.BlockSpec((1, K, BLK, C), lambda b: (b, 0, 0, 0)),
          pl.BlockSpec((K * C, C), lambda b: (0, 0)),
      ],
      out_specs=pl.BlockSpec((BLK, C), lambda b: (b, 0)),
      out_shape=jax.ShapeDtypeStruct((NP, C), jnp.float32),
      scratch_shapes=[pltpu.VMEM((BLK, K * C), jnp.float32)],
      compiler_params=pltpu.CompilerParams(
          dimension_semantics=("parallel",)),
  )(g, wflat)


def _tc_stats(out_pre):
  def body(o_ref, st_ref):
    x = o_ref[...]
    s = jnp.sum(x, axis=0, keepdims=True)
    q = jnp.sum(x * x, axis=0, keepdims=True)
    z = jnp.zeros((1, C), jnp.float32)
    tile = jnp.concatenate(
        [jnp.concatenate([s, z], axis=1),
         jnp.concatenate([q, z], axis=1),
         jnp.zeros((6, 128), jnp.float32)], axis=0)

    @pl.when(pl.program_id(0) == 0)
    def _():
      st_ref[...] = tile

    @pl.when(pl.program_id(0) != 0)
    def _():
      st_ref[...] += tile

  return pl.pallas_call(
      body,
      grid=(NB,),
      in_specs=[pl.BlockSpec((BLK, C), lambda b: (b, 0))],
      out_specs=pl.BlockSpec((8, 128), lambda b: (0, 0)),
      out_shape=jax.ShapeDtypeStruct((8, 128), jnp.float32),
      compiler_params=pltpu.CompilerParams(
          dimension_semantics=("arbitrary",)),
  )(out_pre)


def _tc_bn_relu(out_pre, stats, gamma8, beta8):
  def body(o_ref, st_ref, ga_ref, be_ref, out_ref):
    s = st_ref[0:1, 0:C]
    q = st_ref[1:2, 0:C]
    mean = s * (1.0 / N)
    var = q * (1.0 / N) - mean * mean
    inv = lax.rsqrt(var + 1e-5)
    scale = ga_ref[0:1, :] * inv
    shift = be_ref[0:1, :] - mean * scale
    out_ref[...] = jnp.maximum(o_ref[...] * scale + shift, 0.0)

  return pl.pallas_call(
      body,
      grid=(NB,),
      in_specs=[
          pl.BlockSpec((BLK, C), lambda b: (b, 0)),
          pl.BlockSpec((8, 128), lambda b: (0, 0)),
          pl.BlockSpec((8, C), lambda b: (0, 0)),
          pl.BlockSpec((8, C), lambda b: (0, 0)),
      ],
      out_specs=pl.BlockSpec((BLK, C), lambda b: (b, 0)),
      out_shape=jax.ShapeDtypeStruct((NP, C), jnp.float32),
      compiler_params=pltpu.CompilerParams(
          dimension_semantics=("parallel",)),
  )(out_pre, stats, gamma8, beta8)


def kernel(feats, W, gamma, beta, src, dst):
  feats_pad = jnp.concatenate(
      [feats, jnp.zeros((FP - N, C), jnp.float32)], axis=0)
  src_flat = src.reshape(K * N)
  dst_flat = dst.reshape(K * N)
  wflat = W.reshape(K * C, C)
  gamma8 = jnp.broadcast_to(gamma[None, :], (8, C))
  beta8 = jnp.broadcast_to(beta[None, :], (8, C))

  g = _sc_invert_gather(feats_pad, src_flat, dst_flat)
  out_pre = _tc_gemm(g, wflat)
  stats = _tc_stats(out_pre)
  out = _tc_bn_relu(out_pre, stats, gamma8, beta8)
  return out[:N]


# speedup vs baseline: 1.0465x; 1.0006x over previous
import functools

import jax
import jax.numpy as jnp
from jax import lax
from jax.experimental import pallas as pl
from jax.experimental.pallas import tpu as pltpu
from jax.experimental.pallas import tpu_sc as plsc

N = 100000
C = 64
K = 27
BLK = 1024
NB = 98
NP = NB * BLK
NQ8 = NP // 8
EP = NP + 8
FP = N + 256
ECH = 2000
EB = 400
NEC = N // EB
FSEG = 224
HB = 256
NHB = NP // HB
KS0 = 14
SINK = 2 ** 30
MAXB1 = -(-(KS0 * NEC) // 16)
MAXB2 = -(-(KS0 * NHB) // 16)


def _sc_invert_gather(feats_pad, src, dst):
  mesh = plsc.VectorSubcoreMesh(core_axis_name="c", subcore_axis_name="s")

  @functools.partial(
      pl.kernel,
      out_type=(jax.ShapeDtypeStruct((NB * K * BLK * C,), jnp.float32),
                jax.ShapeDtypeStruct((K * NP,), jnp.int32),
                jax.ShapeDtypeStruct((K * EP * C,), jnp.float32)),
      mesh=mesh,
      compiler_params=pltpu.CompilerParams(
          needs_layout_passes=False, use_tc_tiling_on_sc=False),
      scratch_types=[
          pltpu.VMEM((NQ8,), jnp.int32),
          pltpu.VMEM((ECH,), jnp.int32),
          pltpu.VMEM((EB,), jnp.int32),
          pltpu.VMEM((EB,), jnp.int32),
          [pltpu.VMEM((FSEG * C,), jnp.float32) for _ in range(2)],
          pltpu.VMEM((EB * C,), jnp.float32),
          pltpu.VMEM((HB,), jnp.int32),
          pltpu.VMEM(((HB + 8) * C,), jnp.float32),
          [pltpu.VMEM((HB * C,), jnp.float32) for _ in range(2)],
          [pltpu.SemaphoreType.DMA for _ in range(2)],
          [pltpu.SemaphoreType.DMA for _ in range(2)],
          pltpu.SMEM((32,), jnp.int32),
          pltpu.SMEM((32,), jnp.int32),
      ],
  )
  def sc_kernel(feats_hbm, src_hbm, dst_hbm, g_hbm, inv_hbm, a_hbm,
                inv_v, dbufa, dbufb, sbufb, fbuf, abuf,
                ivbuf, awin, gbuf, lsem, wsem, gmin_s, gmax_s):
    cid = lax.axis_index("c")
    sid = lax.axis_index("s")
    kbase = cid * KS0
    nk = KS0 - cid
    iota = lax.iota(jnp.int32, 16)
    iota_c = iota * C

    @pl.when(sid < nk)
    def _build():
      k = kbase + sid
      for h in range(8):
        lo = h * NQ8

        @pl.loop(0, NQ8 // 16)
        def _init(i):
          inv_v[pl.ds(i * 16, 16)] = jnp.full((16,), SINK, jnp.int32)

        @pl.loop(0, N // ECH)
        def _chunk(j):
          e0 = pl.multiple_of(k * N + j * ECH, 8)
          pltpu.sync_copy(dst_hbm.at[pl.ds(e0, ECH)], dbufa)

          @pl.loop(0, ECH // 16)
          def _scatter(i):
            dv = dbufa[pl.ds(i * 16, 16)]
            ev = jnp.full((16,), j * ECH + i * 16, jnp.int32) + iota
            m = (dv >= lo) & (dv < lo + NQ8) & (dv < N)
            iv = jnp.where(m, dv - lo, jnp.zeros((16,), jnp.int32))
            plsc.store_scatter(inv_v, [iv], ev, mask=m)

        pltpu.sync_copy(
            inv_v, inv_hbm.at[pl.ds(pl.multiple_of(k * NP + lo, 8), NQ8)])

    plsc.subcore_barrier()

    @pl.loop(0, MAXB1)
    def _b1(t):
      j = t * 16 + sid

      @pl.when(j < nk * NEC)
      def _item():
        k_i = j // NEC
        k = kbase + k_i
        e0 = (j - k_i * NEC) * EB
        base = pl.multiple_of(k * N + e0, 8)
        pltpu.sync_copy(dst_hbm.at[pl.ds(base, EB)], dbufb)
        pltpu.sync_copy(src_hbm.at[pl.ds(base, EB)], sbufb)

        def _group_mm(g, carry):
          glo, ghi = carry
          dv = dbufb[pl.ds(g * 16, 16)]
          sv = sbufb[pl.ds(g * 16, 16)]
          m = dv < N
          mn = jnp.min(jnp.where(m, sv, jnp.full((16,), SINK, jnp.int32)))
          mx = jnp.max(jnp.where(m, sv, jnp.full((16,), -1, jnp.int32)))
          gmin_s[g] = mn
          gmax_s[g] = mx
          return (jnp.minimum(glo, mn), jnp.maximum(ghi, mx))

        lo, hi = lax.fori_loop(0, EB // 16, _group_mm,
                               (jnp.int32(SINK), jnp.int32(-1)))

        @pl.when(hi >= 0)
        def _segs():
          seg0 = lo & ~jnp.int32(7)
          nseg = (hi - seg0) // FSEG + 1

          def _load_seg(p, slot):
            sbase = pl.multiple_of((seg0 + p * FSEG) * C, 8)
            pltpu.async_copy(feats_hbm.at[pl.ds(sbase, FSEG * C)],
                             fbuf[slot], lsem[slot])

          _load_seg(0, 0)

          def _seg(p, _):
            sbase = seg0 + p * FSEG
            for slot in range(2):
              @pl.when(lax.rem(p, 2) == slot)
              def _slot(slot=slot):
                pltpu.make_async_copy(feats_hbm.at[pl.ds(0, FSEG * C)],
                                      fbuf[slot], lsem[slot]).wait()

                @pl.when(p + 1 < nseg)
                def _pre():
                  _load_seg(p + 1, 1 - slot)

                def _group(g, _):
                  @pl.when((gmax_s[g] >= sbase)
                           & (gmin_s[g] < sbase + FSEG))
                  def _hit():
                    dv = dbufb[pl.ds(g * 16, 16)]
                    sv = sbufb[pl.ds(g * 16, 16)]
                    m = (dv < N) & (sv >= sbase) & (sv < sbase + FSEG)
                    rel_c = jnp.where(m, (sv - sbase) * C,
                                      jnp.zeros((16,), jnp.int32))
                    rows_c = jnp.full((16,), g * 16 * C, jnp.int32) + iota_c
                    for w in range(C):
                      vals = plsc.load_gather(fbuf[slot], [rel_c + w])
                      plsc.store_scatter(abuf, [rows_c + w], vals, mask=m)
                  return 0

                lax.fori_loop(0, EB // 16, _group, 0)
            return 0

          lax.fori_loop(0, nseg, _seg, 0)

        pltpu.sync_copy(
            abuf,
            a_hbm.at[pl.ds(pl.multiple_of((k * EP + e0) * C, 8), EB * C)])

    plsc.subcore_barrier()

    @pl.loop(0, MAXB2)
    def _b2(t):
      j = t * 16 + sid

      @pl.when(j < nk * NHB)
      def _item():
        k_i = j // NHB
        k = kbase + k_i
        r = j - k_i * NHB
        row0 = r * HB
        b = row0 // BLK
        ri = pl.multiple_of(row0 - b * BLK, 8)
        pltpu.sync_copy(
            inv_hbm.at[pl.ds(pl.multiple_of(k * NP + row0, 8), HB)], ivbuf)

        def _mm(g, carry):
          glo, ghi = carry
          iv = ivbuf[pl.ds(g * 16, 16)]
          m = iv < SINK
          mn = jnp.min(jnp.where(m, iv, jnp.full((16,), SINK, jnp.int32)))
          mx = jnp.max(jnp.where(m, iv, jnp.full((16,), -1, jnp.int32)))
          return (jnp.minimum(glo, mn), jnp.maximum(ghi, mx))

        lo, hi = lax.fori_loop(0, HB // 16, _mm,
                               (jnp.int32(SINK), jnp.int32(-1)))
        lo8 = lo & ~jnp.int32(7)

        @pl.when(hi >= 0)
        def _load():
          pltpu.sync_copy(
              a_hbm.at[pl.ds(pl.multiple_of((k * EP + lo8) * C, 8),
                             (HB + 8) * C)],
              awin)

        for slot in range(2):
          @pl.when(lax.rem(t, 2) == slot)
          def _slot(slot=slot):
            @pl.when(t >= 2)
            def _():
              pltpu.make_async_copy(
                  gbuf[slot], g_hbm.at[pl.ds(0, HB * C)],
                  wsem[slot]).wait()

            @pl.loop(0, HB // 16)
            def _group(g):
              iv = ivbuf[pl.ds(g * 16, 16)]
              m = iv < SINK
              rel_c = jnp.where(m, (iv - lo8) * C,
                                jnp.zeros((16,), jnp.int32))
              rows_c = jnp.full((16,), g * 16 * C, jnp.int32) + iota_c
              for w in range(C):
                vals = plsc.load_gather(awin, [rel_c + w])
                vals = jnp.where(m, vals, jnp.zeros((16,), jnp.float32))
                plsc.store_scatter(gbuf[slot], [rows_c + w], vals)

            goff = pl.multiple_of(((b * K + k) * BLK + ri) * C, 8)
            pltpu.async_copy(gbuf[slot], g_hbm.at[pl.ds(goff, HB * C)],
                             wsem[slot])

    for slot in range(2):
      pltpu.make_async_copy(
          gbuf[slot], g_hbm.at[pl.ds(0, HB * C)], wsem[slot]).wait()

  return sc_kernel(feats_pad, src, dst)[0]


def _tc_gemm(g, wflat):
  def body(g_ref, w_ref, o_ref, x_ref):
    for k in range(K):
      x_ref[:, k * C:(k + 1) * C] = g_ref[0, k, :, :]
    o_ref[...] = jnp.dot(x_ref[...], w_ref[...],
                         preferred_element_type=jnp.float32)

  return pl.pallas_call(
      body,
      grid=(NB,),
      in_specs=[
          pl.BlockSpec((1, K, BLK, C), lambda b: (b, 0, 0, 0)),
          pl.BlockSpec((K * C, C), lambda b: (0, 0)),
      ],
      out_specs=pl.BlockSpec((BLK, C), lambda b: (b, 0)),
      out_shape=jax.ShapeDtypeStruct((NP, C), jnp.float32),
      scratch_shapes=[pltpu.VMEM((BLK, K * C), jnp.float32)],
      compiler_params=pltpu.CompilerParams(
          dimension_semantics=("parallel",)),
  )(g, wflat)


def _tc_stats(out_pre):
  def body(o_ref, st_ref):
    x = o_ref[...]
    s = jnp.sum(x, axis=0, keepdims=True)
    q = jnp.sum(x * x, axis=0, keepdims=True)
    z = jnp.zeros((1, C), jnp.float32)
    tile = jnp.concatenate(
        [jnp.concatenate([s, z], axis=1),
         jnp.concatenate([q, z], axis=1),
         jnp.zeros((6, 128), jnp.float32)], axis=0)

    @pl.when(pl.program_id(0) == 0)
    def _():
      st_ref[...] = tile

    @pl.when(pl.program_id(0) != 0)
    def _():
      st_ref[...] += tile

  return pl.pallas_call(
      body,
      grid=(NB,),
      in_specs=[pl.BlockSpec((BLK, C), lambda b: (b, 0))],
      out_specs=pl.BlockSpec((8, 128), lambda b: (0, 0)),
      out_shape=jax.ShapeDtypeStruct((8, 128), jnp.float32),
      compiler_params=pltpu.CompilerParams(
          dimension_semantics=("arbitrary",)),
  )(out_pre)


def _tc_bn_relu(out_pre, stats, gamma8, beta8):
  def body(o_ref, st_ref, ga_ref, be_ref, out_ref):
    s = st_ref[0:1, 0:C]
    q = st_ref[1:2, 0:C]
    mean = s * (1.0 / N)
    var = q * (1.0 / N) - mean * mean
    inv = lax.rsqrt(var + 1e-5)
    scale = ga_ref[0:1, :] * inv
    shift = be_ref[0:1, :] - mean * scale
    out_ref[...] = jnp.maximum(o_ref[...] * scale + shift, 0.0)

  return pl.pallas_call(
      body,
      grid=(NB,),
      in_specs=[
          pl.BlockSpec((BLK, C), lambda b: (b, 0)),
          pl.BlockSpec((8, 128), lambda b: (0, 0)),
          pl.BlockSpec((8, C), lambda b: (0, 0)),
          pl.BlockSpec((8, C), lambda b: (0, 0)),
      ],
      out_specs=pl.BlockSpec((BLK, C), lambda b: (b, 0)),
      out_shape=jax.ShapeDtypeStruct((NP, C), jnp.float32),
      compiler_params=pltpu.CompilerParams(
          dimension_semantics=("parallel",)),
  )(out_pre, stats, gamma8, beta8)


def kernel(feats, W, gamma, beta, src, dst):
  feats_pad = jnp.concatenate(
      [feats, jnp.zeros((FP - N, C), jnp.float32)], axis=0)
  src_flat = src.reshape(K * N)
  dst_flat = dst.reshape(K * N)
  wflat = W.reshape(K * C, C)
  gamma8 = jnp.broadcast_to(gamma[None, :], (8, C))
  beta8 = jnp.broadcast_to(beta[None, :], (8, C))

  g = _sc_invert_gather(feats_pad.reshape(FP * C), src_flat, dst_flat)
  out_pre = _tc_gemm(g.reshape(NB, K, BLK, C), wflat)
  stats = _tc_stats(out_pre)
  out = _tc_bn_relu(out_pre, stats, gamma8, beta8)
  return out[:N]


# speedup vs baseline: 1.3166x; 1.2581x over previous
import functools

import jax
import jax.numpy as jnp
from jax import lax
from jax.experimental import pallas as pl
from jax.experimental.pallas import tpu as pltpu
from jax.experimental.pallas import tpu_sc as plsc

N = 100000
C = 64
K = 27
BLK = 1024
NB = 98
NP = NB * BLK
NQ8 = NP // 8
EP = NP + 8
FP = N + 256
ECH = 2000
EB = 400
NEC = N // EB
FSEG = 224
HB = 256
NHB = NP // HB
KS0 = 14
SINK = 2 ** 30
MAXB1 = -(-(KS0 * NEC) // 16)
MAXB2 = -(-(KS0 * NHB) // 16)


def _sc_invert_gather(feats_pad, src, dst):
  mesh = plsc.VectorSubcoreMesh(core_axis_name="c", subcore_axis_name="s")

  @functools.partial(
      pl.kernel,
      out_type=(jax.ShapeDtypeStruct((NB * K * BLK * C,), jnp.float32),
                jax.ShapeDtypeStruct((K * NP,), jnp.int32),
                jax.ShapeDtypeStruct((K * EP * C,), jnp.float32)),
      mesh=mesh,
      compiler_params=pltpu.CompilerParams(
          needs_layout_passes=False, use_tc_tiling_on_sc=False),
      scratch_types=[
          pltpu.VMEM((NQ8,), jnp.int32),
          pltpu.VMEM((ECH,), jnp.int32),
          pltpu.VMEM((EB,), jnp.int32),
          pltpu.VMEM((EB,), jnp.int32),
          [pltpu.VMEM((FSEG * C,), jnp.float32) for _ in range(2)],
          pltpu.VMEM((EB * C,), jnp.float32),
          pltpu.VMEM((HB,), jnp.int32),
          pltpu.VMEM(((HB + 8) * C,), jnp.float32),
          [pltpu.VMEM((HB * C,), jnp.float32) for _ in range(2)],
          [pltpu.SemaphoreType.DMA for _ in range(2)],
          [pltpu.SemaphoreType.DMA for _ in range(2)],
          pltpu.SMEM((32,), jnp.int32),
          pltpu.SMEM((32,), jnp.int32),
      ],
  )
  def sc_kernel(feats_hbm, src_hbm, dst_hbm, g_hbm, inv_hbm, a_hbm,
                inv_v, dbufa, dbufb, sbufb, fbuf, abuf,
                ivbuf, awin, gbuf, lsem, wsem, gmin_s, gmax_s):
    cid = lax.axis_index("c")
    sid = lax.axis_index("s")
    kbase = cid * KS0
    nk = KS0 - cid
    iota = lax.iota(jnp.int32, 16)
    iota_c = iota * C

    @pl.when(sid < nk)
    def _build():
      k = kbase + sid
      for h in range(8):
        lo = h * NQ8

        @pl.loop(0, NQ8 // 16)
        def _init(i):
          inv_v[pl.ds(i * 16, 16)] = jnp.full((16,), SINK, jnp.int32)

        @pl.loop(0, N // ECH)
        def _chunk(j):
          e0 = pl.multiple_of(k * N + j * ECH, 8)
          pltpu.sync_copy(dst_hbm.at[pl.ds(e0, ECH)], dbufa)

          @pl.loop(0, ECH // 16)
          def _scatter(i):
            dv = dbufa[pl.ds(i * 16, 16)]
            ev = jnp.full((16,), j * ECH + i * 16, jnp.int32) + iota
            m = (dv >= lo) & (dv < lo + NQ8) & (dv < N)
            iv = jnp.where(m, dv - lo, jnp.zeros((16,), jnp.int32))
            plsc.store_scatter(inv_v, [iv], ev, mask=m)

        pltpu.sync_copy(
            inv_v, inv_hbm.at[pl.ds(pl.multiple_of(k * NP + lo, 8), NQ8)])

    plsc.subcore_barrier()

    @pl.loop(0, MAXB1)
    def _b1(t):
      j = t * 16 + sid

      @pl.when(j < nk * NEC)
      def _item():
        k_i = j // NEC
        k = kbase + k_i
        e0 = (j - k_i * NEC) * EB
        base = pl.multiple_of(k * N + e0, 8)
        pltpu.sync_copy(dst_hbm.at[pl.ds(base, EB)], dbufb)
        pltpu.sync_copy(src_hbm.at[pl.ds(base, EB)], sbufb)

        def _group_mm(g, carry):
          glo, ghi = carry
          dv = dbufb[pl.ds(g * 16, 16)]
          sv = sbufb[pl.ds(g * 16, 16)]
          m = dv < N
          mn = jnp.min(jnp.where(m, sv, jnp.full((16,), SINK, jnp.int32)))
          mx = jnp.max(jnp.where(m, sv, jnp.full((16,), -1, jnp.int32)))
          gmin_s[g] = mn
          gmax_s[g] = mx
          return (jnp.minimum(glo, mn), jnp.maximum(ghi, mx))

        lo, hi = lax.fori_loop(0, EB // 16, _group_mm,
                               (jnp.int32(SINK), jnp.int32(-1)))

        @pl.when(hi >= 0)
        def _segs():
          seg0 = lo & ~jnp.int32(7)
          nseg = (hi - seg0) // FSEG + 1

          def _load_seg(p, slot):
            sbase = pl.multiple_of((seg0 + p * FSEG) * C, 8)
            pltpu.async_copy(feats_hbm.at[pl.ds(sbase, FSEG * C)],
                             fbuf[slot], lsem[slot])

          _load_seg(0, 0)

          def _seg(p, _):
            sbase = seg0 + p * FSEG
            for slot in range(2):
              @pl.when(lax.rem(p, 2) == slot)
              def _slot(slot=slot):
                pltpu.make_async_copy(feats_hbm.at[pl.ds(0, FSEG * C)],
                                      fbuf[slot], lsem[slot]).wait()

                @pl.when(p + 1 < nseg)
                def _pre():
                  _load_seg(p + 1, 1 - slot)

                @plsc.parallel_loop(0, EB // 16, unroll=2)
                def _group(g):
                  @pl.when((gmax_s[g] >= sbase)
                           & (gmin_s[g] < sbase + FSEG))
                  def _hit():
                    dv = dbufb[pl.ds(g * 16, 16)]
                    sv = sbufb[pl.ds(g * 16, 16)]
                    m = (dv < N) & (sv >= sbase) & (sv < sbase + FSEG)
                    rel_c = jnp.where(m, (sv - sbase) * C,
                                      jnp.zeros((16,), jnp.int32))
                    rows_c = jnp.full((16,), g * 16 * C, jnp.int32) + iota_c
                    for w in range(C):
                      vals = plsc.load_gather(fbuf[slot], [rel_c + w])
                      plsc.store_scatter(abuf, [rows_c + w], vals, mask=m)
            return 0

          lax.fori_loop(0, nseg, _seg, 0)

        pltpu.sync_copy(
            abuf,
            a_hbm.at[pl.ds(pl.multiple_of((k * EP + e0) * C, 8), EB * C)])

    plsc.subcore_barrier()

    @pl.loop(0, MAXB2)
    def _b2(t):
      j = t * 16 + sid

      @pl.when(j < nk * NHB)
      def _item():
        k_i = j // NHB
        k = kbase + k_i
        r = j - k_i * NHB
        row0 = r * HB
        b = row0 // BLK
        ri = pl.multiple_of(row0 - b * BLK, 8)
        pltpu.sync_copy(
            inv_hbm.at[pl.ds(pl.multiple_of(k * NP + row0, 8), HB)], ivbuf)

        def _mm(g, carry):
          glo, ghi = carry
          iv = ivbuf[pl.ds(g * 16, 16)]
          m = iv < SINK
          mn = jnp.min(jnp.where(m, iv, jnp.full((16,), SINK, jnp.int32)))
          mx = jnp.max(jnp.where(m, iv, jnp.full((16,), -1, jnp.int32)))
          return (jnp.minimum(glo, mn), jnp.maximum(ghi, mx))

        lo, hi = lax.fori_loop(0, HB // 16, _mm,
                               (jnp.int32(SINK), jnp.int32(-1)))
        lo8 = lo & ~jnp.int32(7)

        @pl.when(hi >= 0)
        def _load():
          pltpu.sync_copy(
              a_hbm.at[pl.ds(pl.multiple_of((k * EP + lo8) * C, 8),
                             (HB + 8) * C)],
              awin)

        for slot in range(2):
          @pl.when(lax.rem(t, 2) == slot)
          def _slot(slot=slot):
            @pl.when(t >= 2)
            def _():
              pltpu.make_async_copy(
                  gbuf[slot], g_hbm.at[pl.ds(0, HB * C)],
                  wsem[slot]).wait()

            @plsc.parallel_loop(0, HB // 16, unroll=2)
            def _group(g):
              iv = ivbuf[pl.ds(g * 16, 16)]
              m = iv < SINK
              rel_c = jnp.where(m, (iv - lo8) * C,
                                jnp.zeros((16,), jnp.int32))
              rows_c = jnp.full((16,), g * 16 * C, jnp.int32) + iota_c
              for w in range(C):
                vals = plsc.load_gather(awin, [rel_c + w])
                vals = jnp.where(m, vals, jnp.zeros((16,), jnp.float32))
                plsc.store_scatter(gbuf[slot], [rows_c + w], vals)

            goff = pl.multiple_of(((b * K + k) * BLK + ri) * C, 8)
            pltpu.async_copy(gbuf[slot], g_hbm.at[pl.ds(goff, HB * C)],
                             wsem[slot])

    for slot in range(2):
      pltpu.make_async_copy(
          gbuf[slot], g_hbm.at[pl.ds(0, HB * C)], wsem[slot]).wait()

  return sc_kernel(feats_pad, src, dst)[0]


def _tc_gemm(g, wflat):
  def body(g_ref, w_ref, o_ref, x_ref):
    for k in range(K):
      x_ref[:, k * C:(k + 1) * C] = g_ref[0, k, :, :]
    o_ref[...] = jnp.dot(x_ref[...], w_ref[...],
                         preferred_element_type=jnp.float32)

  return pl.pallas_call(
      body,
      grid=(NB,),
      in_specs=[
          pl.BlockSpec((1, K, BLK, C), lambda b: (b, 0, 0, 0)),
          pl.BlockSpec((K * C, C), lambda b: (0, 0)),
      ],
      out_specs=pl.BlockSpec((BLK, C), lambda b: (b, 0)),
      out_shape=jax.ShapeDtypeStruct((NP, C), jnp.float32),
      scratch_shapes=[pltpu.VMEM((BLK, K * C), jnp.float32)],
      compiler_params=pltpu.CompilerParams(
          dimension_semantics=("parallel",)),
  )(g, wflat)


def _tc_stats(out_pre):
  def body(o_ref, st_ref):
    x = o_ref[...]
    s = jnp.sum(x, axis=0, keepdims=True)
    q = jnp.sum(x * x, axis=0, keepdims=True)
    z = jnp.zeros((1, C), jnp.float32)
    tile = jnp.concatenate(
        [jnp.concatenate([s, z], axis=1),
         jnp.concatenate([q, z], axis=1),
         jnp.zeros((6, 128), jnp.float32)], axis=0)

    @pl.when(pl.program_id(0) == 0)
    def _():
      st_ref[...] = tile

    @pl.when(pl.program_id(0) != 0)
    def _():
      st_ref[...] += tile

  return pl.pallas_call(
      body,
      grid=(NB,),
      in_specs=[pl.BlockSpec((BLK, C), lambda b: (b, 0))],
      out_specs=pl.BlockSpec((8, 128), lambda b: (0, 0)),
      out_shape=jax.ShapeDtypeStruct((8, 128), jnp.float32),
      compiler_params=pltpu.CompilerParams(
          dimension_semantics=("arbitrary",)),
  )(out_pre)


def _tc_bn_relu(out_pre, stats, gamma8, beta8):
  def body(o_ref, st_ref, ga_ref, be_ref, out_ref):
    s = st_ref[0:1, 0:C]
    q = st_ref[1:2, 0:C]
    mean = s * (1.0 / N)
    var = q * (1.0 / N) - mean * mean
    inv = lax.rsqrt(var + 1e-5)
    scale = ga_ref[0:1, :] * inv
    shift = be_ref[0:1, :] - mean * scale
    out_ref[...] = jnp.maximum(o_ref[...] * scale + shift, 0.0)

  return pl.pallas_call(
      body,
      grid=(NB,),
      in_specs=[
          pl.BlockSpec((BLK, C), lambda b: (b, 0)),
          pl.BlockSpec((8, 128), lambda b: (0, 0)),
          pl.BlockSpec((8, C), lambda b: (0, 0)),
          pl.BlockSpec((8, C), lambda b: (0, 0)),
      ],
      out_specs=pl.BlockSpec((BLK, C), lambda b: (b, 0)),
      out_shape=jax.ShapeDtypeStruct((NP, C), jnp.float32),
      compiler_params=pltpu.CompilerParams(
          dimension_semantics=("parallel",)),
  )(out_pre, stats, gamma8, beta8)


def kernel(feats, W, gamma, beta, src, dst):
  feats_pad = jnp.concatenate(
      [feats, jnp.zeros((FP - N, C), jnp.float32)], axis=0)
  src_flat = src.reshape(K * N)
  dst_flat = dst.reshape(K * N)
  wflat = W.reshape(K * C, C)
  gamma8 = jnp.broadcast_to(gamma[None, :], (8, C))
  beta8 = jnp.broadcast_to(beta[None, :], (8, C))

  g = _sc_invert_gather(feats_pad.reshape(FP * C), src_flat, dst_flat)
  out_pre = _tc_gemm(g.reshape(NB, K, BLK, C), wflat)
  stats = _tc_stats(out_pre)
  out = _tc_bn_relu(out_pre, stats, gamma8, beta8)
  return out[:N]
